# Initial kernel scaffold; baseline (speedup 1.0000x reference)
import jax
import jax.numpy as jnp
from jax.experimental import pallas as pl


def kernel(x, entity_emb, time_emb, num_nodes, params):
    raise NotImplementedError("write your pallas kernel here")


# speedup vs baseline: 1236.8517x; 1236.8517x over previous
import jax
import jax.numpy as jnp
from jax.experimental import pallas as pl

N = 512
INPUT_DIM = 32
EMBED_DIM = 64
HIDDEN = 64
LATENT = 32
PERIODS = 3


def _fwd_kernel(
    x_ref, ent_ref, tim_ref, eps_ref,
    ws_ref, as_ref,
    entW_ref, entb_ref, timW_ref, timb_ref, att_ref,
    ezW_ref, ezb_ref, elzW_ref, elzb_ref,
    ehW_ref, ehb_ref, elhW_ref, elhb_ref,
    muW_ref, mub_ref, lvW_ref, lvb_ref,
    decW_ref, decb_ref,
    dzW_ref, dzb_ref, dlzW_ref, dlzb_ref,
    dhW_ref, dhb_ref, dlhW_ref, dlhb_ref,
    recon_ref, mu_ref, lv_ref, w_ref, a_ref,
):
    ws = ws_ref[...]
    asc = as_ref[...]
    ri = jax.lax.broadcasted_iota(jnp.int32, (N, N), 0)
    ci = jax.lax.broadcasted_iota(jnp.int32, (N, N), 1)
    W = jnp.where(ri == ci, 0.0, jax.nn.sigmoid(ws))
    A = jax.nn.sigmoid(asc)
    w_ref[...] = W
    a_ref[...] = A
    S = W + A

    ones = jnp.ones((N, 1), jnp.float32)
    deg = jax.lax.dot_general(
        S, ones, (((0,), (0,)), ((), ())), preferred_element_type=jnp.float32
    ) + 1.0
    dinv = jax.lax.rsqrt(deg)
    dinv2 = dinv * dinv

    def conv(y, Wg, bg):
        xw = jnp.dot(y, Wg, preferred_element_type=jnp.float32)
        v = dinv * xw
        u = jax.lax.dot_general(
            S, v, (((0,), (0,)), ((), ())), preferred_element_type=jnp.float32
        )
        return dinv * u + dinv2 * xw + bg

    probs = jax.nn.softmax(att_ref[...], axis=-1)

    entW = entW_ref[...]
    entb = entb_ref[...]
    timW = timW_ref[...]
    timb = timb_ref[...]
    ezW = ezW_ref[...]
    ehW = ehW_ref[...]

    Hacc = jnp.zeros((N, HIDDEN), jnp.float32)
    for t in range(PERIODS):
        xt = x_ref[t]
        ent_h = jax.nn.relu(
            jnp.dot(ent_ref[t], entW, preferred_element_type=jnp.float32) + entb)
        tim_h = jax.nn.relu(
            jnp.dot(tim_ref[t], timW, preferred_element_type=jnp.float32) + timb)
        h = jnp.concatenate([xt, ent_h, tim_h], axis=1)
        cz = conv(h, ezW, ezb_ref[...])
        Z = jax.nn.sigmoid(
            jnp.dot(cz, elzW_ref[...], preferred_element_type=jnp.float32)
            + elzb_ref[...])
        ch = conv(h, ehW, ehb_ref[...])
        Ht = jnp.tanh(
            jnp.dot(ch, elhW_ref[...], preferred_element_type=jnp.float32)
            + elhb_ref[...])
        Hacc = Hacc + probs[0, t] * ((1.0 - Z) * Ht)

    enc = jax.nn.relu(Hacc)
    mu = jnp.dot(enc, muW_ref[...], preferred_element_type=jnp.float32) + mub_ref[...]
    lv = jnp.dot(enc, lvW_ref[...], preferred_element_type=jnp.float32) + lvb_ref[...]
    mu_ref[...] = mu
    lv_ref[...] = lv
    z = mu + eps_ref[...] * jnp.exp(0.5 * lv)
    dh = jnp.dot(z, decW_ref[...], preferred_element_type=jnp.float32) + decb_ref[...]

    cz = conv(dh, dzW_ref[...], dzb_ref[...])
    Z = jax.nn.sigmoid(
        jnp.dot(cz, dlzW_ref[...], preferred_element_type=jnp.float32)
        + dlzb_ref[...])
    ch = conv(dh, dhW_ref[...], dhb_ref[...])
    Ht = jnp.tanh(
        jnp.dot(ch, dlhW_ref[...], preferred_element_type=jnp.float32)
        + dlhb_ref[...])
    recon_ref[...] = jax.nn.relu((1.0 - Z) * Ht)


def kernel(x, entity_emb, time_emb, num_nodes, params):
    p = params
    f32 = jnp.float32

    def row(v):
        return jnp.reshape(v, (1, -1)).astype(f32)

    eps = jax.random.normal(jax.random.key(42), (N, LATENT), f32)
    operands = [
        x[0].astype(f32),
        entity_emb[0].astype(f32),
        time_emb[0].astype(f32),
        eps,
        p['W_score'].astype(f32), p['A_score'].astype(f32),
        p['ent_W'].astype(f32), row(p['ent_b']),
        p['time_W'].astype(f32), row(p['time_b']),
        row(p['att']),
        p['e_conv_z_W'].astype(f32), row(p['e_conv_z_b']),
        p['e_lin_z_W'][:HIDDEN].astype(f32), row(p['e_lin_z_b']),
        p['e_conv_h_W'].astype(f32), row(p['e_conv_h_b']),
        p['e_lin_h_W'][:HIDDEN].astype(f32), row(p['e_lin_h_b']),
        p['mu_W'].astype(f32), row(p['mu_b']),
        p['lv_W'].astype(f32), row(p['lv_b']),
        p['dec_W'].astype(f32), row(p['dec_b']),
        p['d_conv_z_W'].astype(f32), row(p['d_conv_z_b']),
        p['d_lin_z_W'][:INPUT_DIM].astype(f32), row(p['d_lin_z_b']),
        p['d_conv_h_W'].astype(f32), row(p['d_conv_h_b']),
        p['d_lin_h_W'][:INPUT_DIM].astype(f32), row(p['d_lin_h_b']),
    ]
    out_shape = (
        jax.ShapeDtypeStruct((N, INPUT_DIM), f32),
        jax.ShapeDtypeStruct((N, LATENT), f32),
        jax.ShapeDtypeStruct((N, LATENT), f32),
        jax.ShapeDtypeStruct((N, N), f32),
        jax.ShapeDtypeStruct((N, N), f32),
    )
    return pl.pallas_call(_fwd_kernel, out_shape=out_shape)(*operands)


# speedup vs baseline: 1355.4131x; 1.0959x over previous
import jax
import jax.numpy as jnp
from jax.experimental import pallas as pl

N = 512
INPUT_DIM = 32
EMBED_DIM = 64
HIDDEN = 64
LATENT = 32
PERIODS = 3

import numpy as _np
_EPS = _np.asarray(
    jax.random.normal(jax.random.key(42), (N, LATENT), jnp.float32))


def _fwd_kernel(
    x_ref, ent_ref, tim_ref, eps_ref,
    ws_ref, as_ref,
    entW_ref, entb_ref, timW_ref, timb_ref, att_ref,
    ezW_ref, ezb_ref, elzW_ref, elzb_ref,
    ehW_ref, ehb_ref, elhW_ref, elhb_ref,
    muW_ref, mub_ref, lvW_ref, lvb_ref,
    decW_ref, decb_ref,
    dzW_ref, dzb_ref, dlzW_ref, dlzb_ref,
    dhW_ref, dhb_ref, dlhW_ref, dlhb_ref,
    recon_ref, mu_ref, lv_ref, w_ref, a_ref,
):
    ws = ws_ref[...]
    asc = as_ref[...]
    ri = jax.lax.broadcasted_iota(jnp.int32, (N, N), 0)
    ci = jax.lax.broadcasted_iota(jnp.int32, (N, N), 1)
    W = jnp.where(ri == ci, 0.0, jax.nn.sigmoid(ws))
    A = jax.nn.sigmoid(asc)
    w_ref[...] = W
    a_ref[...] = A
    S = W + A

    ones = jnp.ones((N, 1), jnp.float32)
    deg = jax.lax.dot_general(
        S, ones, (((0,), (0,)), ((), ())), preferred_element_type=jnp.float32
    ) + 1.0
    dinv = jax.lax.rsqrt(deg)
    dinv2 = dinv * dinv

    def conv(y, Wg, bg):
        xw = jnp.dot(y, Wg, preferred_element_type=jnp.float32)
        v = dinv * xw
        u = jax.lax.dot_general(
            S, v, (((0,), (0,)), ((), ())), preferred_element_type=jnp.float32
        )
        return dinv * u + dinv2 * xw + bg

    probs = jax.nn.softmax(att_ref[...], axis=-1)

    entW = entW_ref[...]
    entb = entb_ref[...]
    timW = timW_ref[...]
    timb = timb_ref[...]
    ezW = ezW_ref[...]
    ehW = ehW_ref[...]

    Hacc = jnp.zeros((N, HIDDEN), jnp.float32)
    for t in range(PERIODS):
        xt = x_ref[t]
        ent_h = jax.nn.relu(
            jnp.dot(ent_ref[t], entW, preferred_element_type=jnp.float32) + entb)
        tim_h = jax.nn.relu(
            jnp.dot(tim_ref[t], timW, preferred_element_type=jnp.float32) + timb)
        h = jnp.concatenate([xt, ent_h, tim_h], axis=1)
        cz = conv(h, ezW, ezb_ref[...])
        Z = jax.nn.sigmoid(
            jnp.dot(cz, elzW_ref[...], preferred_element_type=jnp.float32)
            + elzb_ref[...])
        ch = conv(h, ehW, ehb_ref[...])
        Ht = jnp.tanh(
            jnp.dot(ch, elhW_ref[...], preferred_element_type=jnp.float32)
            + elhb_ref[...])
        Hacc = Hacc + probs[0, t] * ((1.0 - Z) * Ht)

    enc = jax.nn.relu(Hacc)
    mu = jnp.dot(enc, muW_ref[...], preferred_element_type=jnp.float32) + mub_ref[...]
    lv = jnp.dot(enc, lvW_ref[...], preferred_element_type=jnp.float32) + lvb_ref[...]
    mu_ref[...] = mu
    lv_ref[...] = lv
    z = mu + eps_ref[...] * jnp.exp(0.5 * lv)
    dh = jnp.dot(z, decW_ref[...], preferred_element_type=jnp.float32) + decb_ref[...]

    cz = conv(dh, dzW_ref[...], dzb_ref[...])
    Z = jax.nn.sigmoid(
        jnp.dot(cz, dlzW_ref[...], preferred_element_type=jnp.float32)
        + dlzb_ref[...])
    ch = conv(dh, dhW_ref[...], dhb_ref[...])
    Ht = jnp.tanh(
        jnp.dot(ch, dlhW_ref[...], preferred_element_type=jnp.float32)
        + dlhb_ref[...])
    recon_ref[...] = jax.nn.relu((1.0 - Z) * Ht)


def kernel(x, entity_emb, time_emb, num_nodes, params):
    p = params
    f32 = jnp.float32

    def row(v):
        return jnp.reshape(v, (1, -1)).astype(f32)

    eps = jnp.asarray(_EPS)
    operands = [
        x[0].astype(f32),
        entity_emb[0].astype(f32),
        time_emb[0].astype(f32),
        eps,
        p['W_score'].astype(f32), p['A_score'].astype(f32),
        p['ent_W'].astype(f32), row(p['ent_b']),
        p['time_W'].astype(f32), row(p['time_b']),
        row(p['att']),
        p['e_conv_z_W'].astype(f32), row(p['e_conv_z_b']),
        p['e_lin_z_W'][:HIDDEN].astype(f32), row(p['e_lin_z_b']),
        p['e_conv_h_W'].astype(f32), row(p['e_conv_h_b']),
        p['e_lin_h_W'][:HIDDEN].astype(f32), row(p['e_lin_h_b']),
        p['mu_W'].astype(f32), row(p['mu_b']),
        p['lv_W'].astype(f32), row(p['lv_b']),
        p['dec_W'].astype(f32), row(p['dec_b']),
        p['d_conv_z_W'].astype(f32), row(p['d_conv_z_b']),
        p['d_lin_z_W'][:INPUT_DIM].astype(f32), row(p['d_lin_z_b']),
        p['d_conv_h_W'].astype(f32), row(p['d_conv_h_b']),
        p['d_lin_h_W'][:INPUT_DIM].astype(f32), row(p['d_lin_h_b']),
    ]
    out_shape = (
        jax.ShapeDtypeStruct((N, INPUT_DIM), f32),
        jax.ShapeDtypeStruct((N, LATENT), f32),
        jax.ShapeDtypeStruct((N, LATENT), f32),
        jax.ShapeDtypeStruct((N, N), f32),
        jax.ShapeDtypeStruct((N, N), f32),
    )
    return pl.pallas_call(_fwd_kernel, out_shape=out_shape)(*operands)
